# baseline (speedup 1.0000x reference)
import functools

import jax
import jax.numpy as jnp
from jax import lax
from jax.experimental import pallas as pl
from jax.experimental.pallas import tpu as pltpu
from jax.experimental.pallas import tpu_sc as plsc

N = 10000
E = 320000
D = 128
D_OUT = 64

N_PAD = 10240
CHUNK = 128
NC = 2
NS = 16
NW = NC * NS
E_PAD = N_PAD * NW
ROWS_PER_S = N_PAD // NS
PAD_DST = N + 100
NB = 5
DH = D // NC

DEG_CHUNKS = E_PAD // CHUNK // NW
DEG_GROUPS = DEG_CHUNKS // NB

AGG_CHUNKS = E_PAD // CHUNK // NS
AGG_PASSES = 4
PASS_CHUNKS = AGG_CHUNKS // AGG_PASSES
AGG_NB = 4
PASS_GROUPS = PASS_CHUNKS // AGG_NB

_sc_mesh = plsc.VectorSubcoreMesh(core_axis_name="c", subcore_axis_name="s")


@functools.partial(
    pl.kernel,
    mesh=_sc_mesh,
    out_type=jax.ShapeDtypeStruct((NC * N_PAD,), jnp.float32),
    scratch_types=[
        pltpu.VMEM_SHARED((N_PAD,), jnp.float32),
        pltpu.VMEM((DEG_CHUNKS, CHUNK), jnp.int32),
        pltpu.VMEM((CHUNK,), jnp.float32),
        pltpu.VMEM((ROWS_PER_S,), jnp.float32),
    ]
    + [pltpu.SemaphoreType.DMA] * NB,
)
def _deg_sc(dst_hbm, out_hbm, acc, didx, ones, stage, *sems):
    c = lax.axis_index("c")
    s = lax.axis_index("s")
    wid = s * NC + c

    def fill(i, _):
        stage[pl.ds(i * 16, 16)] = jnp.zeros((16,), jnp.float32)
        return 0

    lax.fori_loop(0, ROWS_PER_S // 16, fill, 0)

    def fill1(i, _):
        ones[pl.ds(i * 16, 16)] = jnp.full((16,), 1.0, jnp.float32)
        return 0

    lax.fori_loop(0, CHUNK // 16, fill1, 0)

    pltpu.sync_copy(stage, acc.at[pl.ds(s * ROWS_PER_S, ROWS_PER_S)])
    pltpu.sync_copy(dst_hbm.at[pl.ds(wid * DEG_CHUNKS, DEG_CHUNKS)], didx)
    plsc.subcore_barrier()

    def scat(ch, b):
        return pltpu.async_copy(ones, acc.at[didx.at[ch]], sems[b], add=True)

    for b in range(NB):
        scat(b, b)

    def body(k, _):
        for b in range(NB):
            pltpu.make_async_copy(ones, acc.at[didx.at[0]], sems[b]).wait()
            scat((k + 1) * NB + b, b)
        return 0

    lax.fori_loop(0, DEG_GROUPS - 1, body, 0)
    for b in range(NB):
        pltpu.make_async_copy(ones, acc.at[didx.at[0]], sems[b]).wait()
    plsc.subcore_barrier()

    pltpu.sync_copy(acc.at[pl.ds(s * ROWS_PER_S, ROWS_PER_S)], stage)
    pltpu.sync_copy(stage, out_hbm.at[pl.ds(c * N_PAD + s * ROWS_PER_S, ROWS_PER_S)])


@functools.partial(
    pl.kernel,
    mesh=_sc_mesh,
    out_type=jax.ShapeDtypeStruct((NC * N_PAD, DH), jnp.float32),
    compiler_params=pltpu.CompilerParams(use_tc_tiling_on_sc=False),
    scratch_types=[
        pltpu.VMEM_SHARED((N_PAD, DH), jnp.float32),
        pltpu.VMEM_SHARED((N_PAD, DH), jnp.float32),
        pltpu.VMEM((PASS_CHUNKS, CHUNK), jnp.int32),
        pltpu.VMEM((PASS_CHUNKS, CHUNK), jnp.int32),
    ]
    + [pltpu.VMEM((CHUNK, DH), jnp.float32)] * AGG_NB
    + [pltpu.SemaphoreType.DMA] * (2 * AGG_NB + 1),
)
def _agg_sc(zlo_hbm, zhi_hbm, src_hbm, dst_hbm, out_hbm, acc, zsp, sidx, didx,
            *rest):
    gbuf = rest[:AGG_NB]
    gsem = rest[AGG_NB : 2 * AGG_NB]
    ssem = rest[2 * AGG_NB : 3 * AGG_NB]
    stsem = rest[3 * AGG_NB]
    c = lax.axis_index("c")
    s = lax.axis_index("s")

    @pl.when(c == 0)
    def _():
        pltpu.async_copy(
            zlo_hbm.at[pl.ds(s * ROWS_PER_S, ROWS_PER_S)],
            zsp.at[pl.ds(s * ROWS_PER_S, ROWS_PER_S)],
            stsem,
        )

    @pl.when(c == 1)
    def _():
        pltpu.async_copy(
            zhi_hbm.at[pl.ds(s * ROWS_PER_S, ROWS_PER_S)],
            zsp.at[pl.ds(s * ROWS_PER_S, ROWS_PER_S)],
            stsem,
        )

    def fillz(i, _):
        gbuf[0][i // (DH // 16), pl.ds((i % (DH // 16)) * 16, 16)] = jnp.zeros(
            (16,), jnp.float32
        )
        return 0

    lax.fori_loop(0, CHUNK * DH // 16, fillz, 0)

    def zrow(j, _):
        pltpu.sync_copy(gbuf[0], acc.at[pl.ds(s * ROWS_PER_S + j * CHUNK, CHUNK)])
        return 0

    lax.fori_loop(0, ROWS_PER_S // CHUNK, zrow, 0)
    pltpu.make_async_copy(
        zlo_hbm.at[pl.ds(0, ROWS_PER_S)],
        zsp.at[pl.ds(0, ROWS_PER_S)],
        stsem,
    ).wait()
    plsc.subcore_barrier()

    def gat(ch, b):
        return pltpu.async_copy(zsp.at[sidx.at[ch]], gbuf[b], gsem[b])

    def gat_wait(b):
        pltpu.make_async_copy(zsp.at[sidx.at[0]], gbuf[b], gsem[b]).wait()

    def scat(ch, b):
        return pltpu.async_copy(gbuf[b], acc.at[didx.at[ch]], ssem[b], add=True)

    def scat_wait(b):
        pltpu.make_async_copy(gbuf[b], acc.at[didx.at[0]], ssem[b]).wait()

    for p in range(AGG_PASSES):
        base = s * AGG_CHUNKS + p * PASS_CHUNKS
        pltpu.sync_copy(src_hbm.at[pl.ds(base, PASS_CHUNKS)], sidx)
        pltpu.sync_copy(dst_hbm.at[pl.ds(base, PASS_CHUNKS)], didx)

        for b in range(AGG_NB):
            gat(b, b)

        def body(k, _):
            for b in range(AGG_NB):
                gat_wait(b)
                scat(k * AGG_NB + b, b)
            for b in range(AGG_NB):
                scat_wait(b)
                gat((k + 1) * AGG_NB + b, b)
            return 0

        lax.fori_loop(0, PASS_GROUPS - 1, body, 0)

        for b in range(AGG_NB):
            gat_wait(b)
            scat((PASS_GROUPS - 1) * AGG_NB + b, b)
        for b in range(AGG_NB):
            scat_wait(b)
    plsc.subcore_barrier()

    def crow(j, _):
        row = s * ROWS_PER_S + j * CHUNK
        pltpu.sync_copy(acc.at[pl.ds(row, CHUNK)], gbuf[0])
        pltpu.sync_copy(gbuf[0], out_hbm.at[pl.ds(c * N_PAD + row, CHUNK)])
        return 0

    lax.fori_loop(0, ROWS_PER_S // CHUNK, crow, 0)


_RB = 1024
_GRID = N_PAD // _RB


def _k_scale_matmul(x_ref, w_ref, dis_ref, o0_ref, o1_ref):
    z = dis_ref[...] * jnp.dot(
        x_ref[...], w_ref[...], preferred_element_type=jnp.float32
    )
    o0_ref[...] = z[:, :DH]
    o1_ref[...] = z[:, DH:]


def _k_mid(p0_ref, p1_ref, z0_ref, z1_ref, dis_ref, s_ref, u_ref, w_ref,
           o0_ref, o1_ref):
    dis = dis_ref[...]
    p = jnp.concatenate([p0_ref[...], p1_ref[...]], axis=1)
    z = jnp.concatenate([z0_ref[...], z1_ref[...]], axis=1)
    h = jnp.maximum(dis * (p + z) * s_ref[...] + u_ref[...], 0.0)
    zn = dis * jnp.dot(h, w_ref[...], preferred_element_type=jnp.float32)
    o0_ref[...] = zn[:, :DH]
    o1_ref[...] = zn[:, DH:]


def _k_final(p0_ref, p1_ref, z0_ref, z1_ref, dis_ref, s_ref, u_ref, w_ref,
             b_ref, o_ref):
    dis = dis_ref[...]
    p = jnp.concatenate([p0_ref[...], p1_ref[...]], axis=1)
    z = jnp.concatenate([z0_ref[...], z1_ref[...]], axis=1)
    h = jnp.maximum(dis * (p + z) * s_ref[...] + u_ref[...], 0.0)
    o_ref[...] = (
        jnp.dot(h, w_ref[...], preferred_element_type=jnp.float32) + b_ref[...]
    )


def _row_spec(cols):
    return pl.BlockSpec((_RB, cols), lambda i: (i, 0))


def _p1_spec():
    return pl.BlockSpec((_RB, DH), lambda i: (_GRID + i, 0))


def _full_spec(rows, cols):
    return pl.BlockSpec((rows, cols), lambda i: (0, 0))


_ZSHAPE = (
    jax.ShapeDtypeStruct((N_PAD, DH), jnp.float32),
    jax.ShapeDtypeStruct((N_PAD, DH), jnp.float32),
)


def _scale_matmul(x, w, dis_col):
    return pl.pallas_call(
        _k_scale_matmul,
        grid=(_GRID,),
        in_specs=[_row_spec(D), _full_spec(D, D), _row_spec(1)],
        out_specs=(_row_spec(DH), _row_spec(DH)),
        out_shape=_ZSHAPE,
    )(x, w, dis_col)


def _mid(p, z0, z1, dis_col, s_row, u_row, w):
    return pl.pallas_call(
        _k_mid,
        grid=(_GRID,),
        in_specs=[
            _row_spec(DH),
            _p1_spec(),
            _row_spec(DH),
            _row_spec(DH),
            _row_spec(1),
            _full_spec(1, D),
            _full_spec(1, D),
            _full_spec(D, D),
        ],
        out_specs=(_row_spec(DH), _row_spec(DH)),
        out_shape=_ZSHAPE,
    )(p, p, z0, z1, dis_col, s_row, u_row, w)


def _final(p, z0, z1, dis_col, s_row, u_row, w, b_row):
    return pl.pallas_call(
        _k_final,
        grid=(_GRID,),
        in_specs=[
            _row_spec(DH),
            _p1_spec(),
            _row_spec(DH),
            _row_spec(DH),
            _row_spec(1),
            _full_spec(1, D),
            _full_spec(1, D),
            _full_spec(D, D_OUT),
            _full_spec(1, D_OUT),
        ],
        out_specs=pl.BlockSpec((_RB, D_OUT), lambda i: (i, 0)),
        out_shape=jax.ShapeDtypeStruct((N_PAD, D_OUT), jnp.float32),
    )(p, p, z0, z1, dis_col, s_row, u_row, w, b_row)


@jax.jit
def kernel(x, edge_index, W1, b1, g1, be1, rm1, rv1, W2, b2, g2, be2, rm2, rv2,
           Wro, bro):
    pad_e = E_PAD - E
    src_p = jnp.concatenate([edge_index[0], jnp.zeros((pad_e,), jnp.int32)])
    dst_p = jnp.concatenate(
        [edge_index[1], jnp.full((pad_e,), PAD_DST, jnp.int32)]
    )
    src2d = src_p.reshape(E_PAD // CHUNK, CHUNK)
    dst2d = dst_p.reshape(E_PAD // CHUNK, CHUNK)
    x_p = jnp.concatenate([x, jnp.zeros((N_PAD - N, D), jnp.float32)])

    s1 = g1 * lax.rsqrt(rv1 + 1e-5)
    u1 = (b1 - rm1) * s1 + be1
    s2 = g2 * lax.rsqrt(rv2 + 1e-5)
    u2 = (b2 - rm2) * s2 + be2
    s1r, u1r = s1[None, :], u1[None, :]
    s2r, u2r = s2[None, :], u2[None, :]
    bror = bro[None, :]

    degp = _deg_sc(dst2d).reshape(NC, N_PAD)
    dis_col = lax.rsqrt(degp[0] + degp[1] + 1.0)[:, None]

    z1lo, z1hi = _scale_matmul(x_p, W1, dis_col)
    p1_ = _agg_sc(z1lo, z1hi, src2d, dst2d)

    z2lo, z2hi = _mid(p1_, z1lo, z1hi, dis_col, s1r, u1r, W2)
    p2_ = _agg_sc(z2lo, z2hi, src2d, dst2d)

    out = _final(p2_, z2lo, z2hi, dis_col, s2r, u2r, Wro, bror)
    return out[:N]


# speedup vs baseline: 1.0903x; 1.0903x over previous
import functools

import jax
import jax.numpy as jnp
from jax import lax
from jax.experimental import pallas as pl
from jax.experimental.pallas import tpu as pltpu
from jax.experimental.pallas import tpu_sc as plsc

N = 10000
E = 320000
D = 128
D_OUT = 64

N_PAD = 10240
CHUNK = 128
NC = 2
NS = 16
NW = NC * NS
E_PAD = N_PAD * NW
ROWS_PER_S = N_PAD // NS
PAD_DST = N + 100
NB = 5
DH = D // NC

DEG_CHUNKS = E_PAD // CHUNK // NW
DEG_GROUPS = DEG_CHUNKS // NB

AGG_CHUNKS = E_PAD // CHUNK // NS
AGG_PASSES = 2
PASS_CHUNKS = AGG_CHUNKS // AGG_PASSES
AGG_NB = 2
PASS_GROUPS = PASS_CHUNKS // AGG_NB

_sc_mesh = plsc.VectorSubcoreMesh(core_axis_name="c", subcore_axis_name="s")


@functools.partial(
    pl.kernel,
    mesh=_sc_mesh,
    out_type=jax.ShapeDtypeStruct((NC * N_PAD,), jnp.float32),
    scratch_types=[
        pltpu.VMEM_SHARED((N_PAD,), jnp.float32),
        pltpu.VMEM((DEG_CHUNKS, CHUNK), jnp.int32),
        pltpu.VMEM((CHUNK,), jnp.float32),
        pltpu.VMEM((ROWS_PER_S,), jnp.float32),
    ]
    + [pltpu.SemaphoreType.DMA] * NB,
)
def _deg_sc(dst_hbm, out_hbm, acc, didx, ones, stage, *sems):
    c = lax.axis_index("c")
    s = lax.axis_index("s")
    wid = s * NC + c

    def fill(i, _):
        stage[pl.ds(i * 16, 16)] = jnp.zeros((16,), jnp.float32)
        return 0

    lax.fori_loop(0, ROWS_PER_S // 16, fill, 0)

    def fill1(i, _):
        ones[pl.ds(i * 16, 16)] = jnp.full((16,), 1.0, jnp.float32)
        return 0

    lax.fori_loop(0, CHUNK // 16, fill1, 0)

    pltpu.sync_copy(stage, acc.at[pl.ds(s * ROWS_PER_S, ROWS_PER_S)])
    pltpu.sync_copy(dst_hbm.at[pl.ds(wid * DEG_CHUNKS, DEG_CHUNKS)], didx)
    plsc.subcore_barrier()

    def scat(ch, b):
        return pltpu.async_copy(ones, acc.at[didx.at[ch]], sems[b], add=True)

    for b in range(NB):
        scat(b, b)

    def body(k, _):
        for b in range(NB):
            pltpu.make_async_copy(ones, acc.at[didx.at[0]], sems[b]).wait()
            scat((k + 1) * NB + b, b)
        return 0

    lax.fori_loop(0, DEG_GROUPS - 1, body, 0)
    for b in range(NB):
        pltpu.make_async_copy(ones, acc.at[didx.at[0]], sems[b]).wait()
    plsc.subcore_barrier()

    pltpu.sync_copy(acc.at[pl.ds(s * ROWS_PER_S, ROWS_PER_S)], stage)
    pltpu.sync_copy(stage, out_hbm.at[pl.ds(c * N_PAD + s * ROWS_PER_S, ROWS_PER_S)])


@functools.partial(
    pl.kernel,
    mesh=_sc_mesh,
    out_type=jax.ShapeDtypeStruct((NC * N_PAD, DH), jnp.float32),
    compiler_params=pltpu.CompilerParams(use_tc_tiling_on_sc=False),
    scratch_types=[
        pltpu.VMEM_SHARED((N_PAD, DH), jnp.float32),
        pltpu.VMEM_SHARED((N_PAD, DH), jnp.float32),
        pltpu.VMEM((PASS_CHUNKS, CHUNK), jnp.int32),
        pltpu.VMEM((PASS_CHUNKS, CHUNK), jnp.int32),
    ]
    + [pltpu.VMEM((CHUNK, DH), jnp.float32)] * AGG_NB
    + [pltpu.SemaphoreType.DMA] * (2 * AGG_NB + 1),
)
def _agg_sc(zlo_hbm, zhi_hbm, src_hbm, dst_hbm, out_hbm, acc, zsp, sidx, didx,
            *rest):
    gbuf = rest[:AGG_NB]
    gsem = rest[AGG_NB : 2 * AGG_NB]
    ssem = rest[2 * AGG_NB : 3 * AGG_NB]
    stsem = rest[3 * AGG_NB]
    c = lax.axis_index("c")
    s = lax.axis_index("s")

    @pl.when(c == 0)
    def _():
        pltpu.async_copy(
            zlo_hbm.at[pl.ds(s * ROWS_PER_S, ROWS_PER_S)],
            zsp.at[pl.ds(s * ROWS_PER_S, ROWS_PER_S)],
            stsem,
        )

    @pl.when(c == 1)
    def _():
        pltpu.async_copy(
            zhi_hbm.at[pl.ds(s * ROWS_PER_S, ROWS_PER_S)],
            zsp.at[pl.ds(s * ROWS_PER_S, ROWS_PER_S)],
            stsem,
        )

    def fillz(i, _):
        gbuf[0][i // (DH // 16), pl.ds((i % (DH // 16)) * 16, 16)] = jnp.zeros(
            (16,), jnp.float32
        )
        return 0

    lax.fori_loop(0, CHUNK * DH // 16, fillz, 0)

    def zrow(j, _):
        pltpu.sync_copy(gbuf[0], acc.at[pl.ds(s * ROWS_PER_S + j * CHUNK, CHUNK)])
        return 0

    lax.fori_loop(0, ROWS_PER_S // CHUNK, zrow, 0)
    pltpu.make_async_copy(
        zlo_hbm.at[pl.ds(0, ROWS_PER_S)],
        zsp.at[pl.ds(0, ROWS_PER_S)],
        stsem,
    ).wait()
    plsc.subcore_barrier()

    def gat(ch, b):
        return pltpu.async_copy(zsp.at[sidx.at[ch]], gbuf[b], gsem[b])

    def gat_wait(b):
        pltpu.make_async_copy(zsp.at[sidx.at[0]], gbuf[b], gsem[b]).wait()

    def scat(ch, b):
        return pltpu.async_copy(gbuf[b], acc.at[didx.at[ch]], ssem[b], add=True)

    def scat_wait(b):
        pltpu.make_async_copy(gbuf[b], acc.at[didx.at[0]], ssem[b]).wait()

    for p in range(AGG_PASSES):
        base = s * AGG_CHUNKS + p * PASS_CHUNKS
        pltpu.sync_copy(src_hbm.at[pl.ds(base, PASS_CHUNKS)], sidx)
        pltpu.sync_copy(dst_hbm.at[pl.ds(base, PASS_CHUNKS)], didx)

        for b in range(AGG_NB):
            gat(b, b)

        def body(k, _):
            for b in range(AGG_NB):
                gat_wait(b)
                scat(k * AGG_NB + b, b)
            for b in range(AGG_NB):
                scat_wait(b)
                gat((k + 1) * AGG_NB + b, b)
            return 0

        lax.fori_loop(0, PASS_GROUPS - 1, body, 0)

        for b in range(AGG_NB):
            gat_wait(b)
            scat((PASS_GROUPS - 1) * AGG_NB + b, b)
        for b in range(AGG_NB):
            scat_wait(b)
    plsc.subcore_barrier()

    def crow(j, _):
        row = s * ROWS_PER_S + j * CHUNK
        pltpu.sync_copy(acc.at[pl.ds(row, CHUNK)], gbuf[0])
        pltpu.sync_copy(gbuf[0], out_hbm.at[pl.ds(c * N_PAD + row, CHUNK)])
        return 0

    lax.fori_loop(0, ROWS_PER_S // CHUNK, crow, 0)


_RB = 1024
_GRID = N_PAD // _RB


def _k_scale_matmul(x_ref, w_ref, dis_ref, o0_ref, o1_ref):
    z = dis_ref[...] * jnp.dot(
        x_ref[...], w_ref[...], preferred_element_type=jnp.float32
    )
    o0_ref[...] = z[:, :DH]
    o1_ref[...] = z[:, DH:]


def _k_mid(p0_ref, p1_ref, z0_ref, z1_ref, dis_ref, s_ref, u_ref, w_ref,
           o0_ref, o1_ref):
    dis = dis_ref[...]
    p = jnp.concatenate([p0_ref[...], p1_ref[...]], axis=1)
    z = jnp.concatenate([z0_ref[...], z1_ref[...]], axis=1)
    h = jnp.maximum(dis * (p + z) * s_ref[...] + u_ref[...], 0.0)
    zn = dis * jnp.dot(h, w_ref[...], preferred_element_type=jnp.float32)
    o0_ref[...] = zn[:, :DH]
    o1_ref[...] = zn[:, DH:]


def _k_final(p0_ref, p1_ref, z0_ref, z1_ref, dis_ref, s_ref, u_ref, w_ref,
             b_ref, o_ref):
    dis = dis_ref[...]
    p = jnp.concatenate([p0_ref[...], p1_ref[...]], axis=1)
    z = jnp.concatenate([z0_ref[...], z1_ref[...]], axis=1)
    h = jnp.maximum(dis * (p + z) * s_ref[...] + u_ref[...], 0.0)
    o_ref[...] = (
        jnp.dot(h, w_ref[...], preferred_element_type=jnp.float32) + b_ref[...]
    )


def _row_spec(cols):
    return pl.BlockSpec((_RB, cols), lambda i: (i, 0))


def _p1_spec():
    return pl.BlockSpec((_RB, DH), lambda i: (_GRID + i, 0))


def _full_spec(rows, cols):
    return pl.BlockSpec((rows, cols), lambda i: (0, 0))


_ZSHAPE = (
    jax.ShapeDtypeStruct((N_PAD, DH), jnp.float32),
    jax.ShapeDtypeStruct((N_PAD, DH), jnp.float32),
)


def _scale_matmul(x, w, dis_col):
    return pl.pallas_call(
        _k_scale_matmul,
        grid=(_GRID,),
        in_specs=[_row_spec(D), _full_spec(D, D), _row_spec(1)],
        out_specs=(_row_spec(DH), _row_spec(DH)),
        out_shape=_ZSHAPE,
    )(x, w, dis_col)


def _mid(p, z0, z1, dis_col, s_row, u_row, w):
    return pl.pallas_call(
        _k_mid,
        grid=(_GRID,),
        in_specs=[
            _row_spec(DH),
            _p1_spec(),
            _row_spec(DH),
            _row_spec(DH),
            _row_spec(1),
            _full_spec(1, D),
            _full_spec(1, D),
            _full_spec(D, D),
        ],
        out_specs=(_row_spec(DH), _row_spec(DH)),
        out_shape=_ZSHAPE,
    )(p, p, z0, z1, dis_col, s_row, u_row, w)


def _final(p, z0, z1, dis_col, s_row, u_row, w, b_row):
    return pl.pallas_call(
        _k_final,
        grid=(_GRID,),
        in_specs=[
            _row_spec(DH),
            _p1_spec(),
            _row_spec(DH),
            _row_spec(DH),
            _row_spec(1),
            _full_spec(1, D),
            _full_spec(1, D),
            _full_spec(D, D_OUT),
            _full_spec(1, D_OUT),
        ],
        out_specs=pl.BlockSpec((_RB, D_OUT), lambda i: (i, 0)),
        out_shape=jax.ShapeDtypeStruct((N_PAD, D_OUT), jnp.float32),
    )(p, p, z0, z1, dis_col, s_row, u_row, w, b_row)


@jax.jit
def kernel(x, edge_index, W1, b1, g1, be1, rm1, rv1, W2, b2, g2, be2, rm2, rv2,
           Wro, bro):
    pad_e = E_PAD - E
    src_p = jnp.concatenate([edge_index[0], jnp.zeros((pad_e,), jnp.int32)])
    dst_p = jnp.concatenate(
        [edge_index[1], jnp.full((pad_e,), PAD_DST, jnp.int32)]
    )
    src2d = src_p.reshape(E_PAD // CHUNK, CHUNK)
    dst2d = dst_p.reshape(E_PAD // CHUNK, CHUNK)
    x_p = jnp.concatenate([x, jnp.zeros((N_PAD - N, D), jnp.float32)])

    s1 = g1 * lax.rsqrt(rv1 + 1e-5)
    u1 = (b1 - rm1) * s1 + be1
    s2 = g2 * lax.rsqrt(rv2 + 1e-5)
    u2 = (b2 - rm2) * s2 + be2
    s1r, u1r = s1[None, :], u1[None, :]
    s2r, u2r = s2[None, :], u2[None, :]
    bror = bro[None, :]

    degp = _deg_sc(dst2d).reshape(NC, N_PAD)
    dis_col = lax.rsqrt(degp[0] + degp[1] + 1.0)[:, None]

    z1lo, z1hi = _scale_matmul(x_p, W1, dis_col)
    p1_ = _agg_sc(z1lo, z1hi, src2d, dst2d)

    z2lo, z2hi = _mid(p1_, z1lo, z1hi, dis_col, s1r, u1r, W2)
    p2_ = _agg_sc(z2lo, z2hi, src2d, dst2d)

    out = _final(p2_, z2lo, z2hi, dis_col, s2r, u2r, Wro, bror)
    return out[:N]


# speedup vs baseline: 1.1015x; 1.0103x over previous
import functools

import jax
import jax.numpy as jnp
from jax import lax
from jax.experimental import pallas as pl
from jax.experimental.pallas import tpu as pltpu
from jax.experimental.pallas import tpu_sc as plsc

N = 10000
E = 320000
D = 128
D_OUT = 64

N_PAD = 10240
CHUNK = 128
NC = 2
NS = 16
NW = NC * NS
E_PAD = N_PAD * NW
ROWS_PER_S = N_PAD // NS
PAD_DST = N + 100
NB = 5
DH = D // NC

DEG_CHUNKS = E_PAD // CHUNK // NW
DEG_GROUPS = DEG_CHUNKS // NB

AGG_CHUNKS = E_PAD // CHUNK // NS
AGG_PASSES = 2
PASS_CHUNKS = AGG_CHUNKS // AGG_PASSES
AGG_NB = 2
PASS_GROUPS = PASS_CHUNKS // AGG_NB

_sc_mesh = plsc.VectorSubcoreMesh(core_axis_name="c", subcore_axis_name="s")


@functools.partial(
    pl.kernel,
    mesh=_sc_mesh,
    out_type=jax.ShapeDtypeStruct((NC * N_PAD,), jnp.float32),
    scratch_types=[
        pltpu.VMEM_SHARED((N_PAD,), jnp.float32),
        pltpu.VMEM((DEG_CHUNKS, CHUNK), jnp.int32),
        pltpu.VMEM((CHUNK,), jnp.float32),
        pltpu.VMEM((ROWS_PER_S,), jnp.float32),
    ]
    + [pltpu.SemaphoreType.DMA] * NB,
)
def _deg_sc(dst_hbm, out_hbm, acc, didx, ones, stage, *sems):
    c = lax.axis_index("c")
    s = lax.axis_index("s")
    wid = s * NC + c

    def fill(i, _):
        stage[pl.ds(i * 16, 16)] = jnp.zeros((16,), jnp.float32)
        return 0

    lax.fori_loop(0, ROWS_PER_S // 16, fill, 0)

    def fill1(i, _):
        ones[pl.ds(i * 16, 16)] = jnp.full((16,), 1.0, jnp.float32)
        return 0

    lax.fori_loop(0, CHUNK // 16, fill1, 0)

    pltpu.sync_copy(stage, acc.at[pl.ds(s * ROWS_PER_S, ROWS_PER_S)])
    pltpu.sync_copy(dst_hbm.at[pl.ds(wid * DEG_CHUNKS, DEG_CHUNKS)], didx)
    plsc.subcore_barrier()

    def scat(ch, b):
        return pltpu.async_copy(ones, acc.at[didx.at[ch]], sems[b], add=True)

    for b in range(NB):
        scat(b, b)

    def body(k, _):
        for b in range(NB):
            pltpu.make_async_copy(ones, acc.at[didx.at[0]], sems[b]).wait()
            scat((k + 1) * NB + b, b)
        return 0

    lax.fori_loop(0, DEG_GROUPS - 1, body, 0)
    for b in range(NB):
        pltpu.make_async_copy(ones, acc.at[didx.at[0]], sems[b]).wait()
    plsc.subcore_barrier()

    pltpu.sync_copy(acc.at[pl.ds(s * ROWS_PER_S, ROWS_PER_S)], stage)
    pltpu.sync_copy(stage, out_hbm.at[pl.ds(c * N_PAD + s * ROWS_PER_S, ROWS_PER_S)])


@functools.partial(
    pl.kernel,
    mesh=_sc_mesh,
    out_type=jax.ShapeDtypeStruct((NC * N_PAD, DH), jnp.float32),
    compiler_params=pltpu.CompilerParams(use_tc_tiling_on_sc=False),
    scratch_types=[
        pltpu.VMEM_SHARED((N_PAD, DH), jnp.float32),
        pltpu.VMEM_SHARED((N_PAD, DH), jnp.float32),
        pltpu.VMEM((PASS_CHUNKS, CHUNK), jnp.int32),
        pltpu.VMEM((PASS_CHUNKS, CHUNK), jnp.int32),
    ]
    + [pltpu.VMEM((CHUNK, DH), jnp.float32)] * AGG_NB
    + [pltpu.SemaphoreType.DMA] * (2 * AGG_NB + 1),
)
def _agg_sc(zlo_hbm, zhi_hbm, src_hbm, dst_hbm, out_hbm, acc, zsp, sidx, didx,
            *rest):
    gbuf = rest[:AGG_NB]
    gsem = rest[AGG_NB : 2 * AGG_NB]
    ssem = rest[2 * AGG_NB : 3 * AGG_NB]
    stsem = rest[3 * AGG_NB]
    c = lax.axis_index("c")
    s = lax.axis_index("s")

    @pl.when(c == 0)
    def _():
        pltpu.async_copy(
            zlo_hbm.at[pl.ds(s * ROWS_PER_S, ROWS_PER_S)],
            zsp.at[pl.ds(s * ROWS_PER_S, ROWS_PER_S)],
            stsem,
        )

    @pl.when(c == 1)
    def _():
        pltpu.async_copy(
            zhi_hbm.at[pl.ds(s * ROWS_PER_S, ROWS_PER_S)],
            zsp.at[pl.ds(s * ROWS_PER_S, ROWS_PER_S)],
            stsem,
        )

    def fillz(i, _):
        gbuf[0][i // (DH // 16), pl.ds((i % (DH // 16)) * 16, 16)] = jnp.zeros(
            (16,), jnp.float32
        )
        return 0

    lax.fori_loop(0, CHUNK * DH // 16, fillz, 0)

    def zrow(j, _):
        pltpu.sync_copy(gbuf[0], acc.at[pl.ds(s * ROWS_PER_S + j * CHUNK, CHUNK)])
        return 0

    lax.fori_loop(0, ROWS_PER_S // CHUNK, zrow, 0)
    pltpu.make_async_copy(
        zlo_hbm.at[pl.ds(0, ROWS_PER_S)],
        zsp.at[pl.ds(0, ROWS_PER_S)],
        stsem,
    ).wait()
    plsc.subcore_barrier()

    def gat(ch, b):
        return pltpu.async_copy(zsp.at[sidx.at[ch]], gbuf[b], gsem[b])

    def gat_wait(b):
        pltpu.make_async_copy(zsp.at[sidx.at[0]], gbuf[b], gsem[b]).wait()

    def scat(ch, b):
        return pltpu.async_copy(gbuf[b], acc.at[didx.at[ch]], ssem[b], add=True)

    def scat_wait(b):
        pltpu.make_async_copy(gbuf[b], acc.at[didx.at[0]], ssem[b]).wait()

    for p in range(AGG_PASSES):
        base = s * AGG_CHUNKS + p * PASS_CHUNKS
        pltpu.sync_copy(src_hbm.at[pl.ds(base, PASS_CHUNKS)], sidx)
        pltpu.sync_copy(dst_hbm.at[pl.ds(base, PASS_CHUNKS)], didx)

        for b in range(AGG_NB):
            gat(b, b)

        def body(k, _):
            for b in range(AGG_NB):
                gat_wait(b)
                scat(k * AGG_NB + b, b)
            for b in range(AGG_NB):
                scat_wait(b)
                gat((k + 1) * AGG_NB + b, b)
            return 0

        lax.fori_loop(0, PASS_GROUPS - 1, body, 0)

        for b in range(AGG_NB):
            gat_wait(b)
            scat((PASS_GROUPS - 1) * AGG_NB + b, b)
        for b in range(AGG_NB):
            scat_wait(b)
    plsc.subcore_barrier()

    def crow(j, _):
        row = s * ROWS_PER_S + j * CHUNK
        pltpu.sync_copy(acc.at[pl.ds(row, CHUNK)], gbuf[0])
        pltpu.sync_copy(gbuf[0], out_hbm.at[pl.ds(c * N_PAD + row, CHUNK)])
        return 0

    lax.fori_loop(0, ROWS_PER_S // CHUNK, crow, 0)


_RB = 2048
_GRID = N_PAD // _RB


def _k_scale_matmul(x_ref, w_ref, dis_ref, o0_ref, o1_ref):
    z = dis_ref[...] * jnp.dot(
        x_ref[...], w_ref[...], preferred_element_type=jnp.float32
    )
    o0_ref[...] = z[:, :DH]
    o1_ref[...] = z[:, DH:]


def _k_mid(p0_ref, p1_ref, z0_ref, z1_ref, dis_ref, s_ref, u_ref, w_ref,
           o0_ref, o1_ref):
    dis = dis_ref[...]
    p = jnp.concatenate([p0_ref[...], p1_ref[...]], axis=1)
    z = jnp.concatenate([z0_ref[...], z1_ref[...]], axis=1)
    h = jnp.maximum(dis * (p + z) * s_ref[...] + u_ref[...], 0.0)
    zn = dis * jnp.dot(h, w_ref[...], preferred_element_type=jnp.float32)
    o0_ref[...] = zn[:, :DH]
    o1_ref[...] = zn[:, DH:]


def _k_final(p0_ref, p1_ref, z0_ref, z1_ref, dis_ref, s_ref, u_ref, w_ref,
             b_ref, o_ref):
    dis = dis_ref[...]
    p = jnp.concatenate([p0_ref[...], p1_ref[...]], axis=1)
    z = jnp.concatenate([z0_ref[...], z1_ref[...]], axis=1)
    h = jnp.maximum(dis * (p + z) * s_ref[...] + u_ref[...], 0.0)
    o_ref[...] = (
        jnp.dot(h, w_ref[...], preferred_element_type=jnp.float32) + b_ref[...]
    )


def _row_spec(cols):
    return pl.BlockSpec((_RB, cols), lambda i: (i, 0))


def _p1_spec():
    return pl.BlockSpec((_RB, DH), lambda i: (_GRID + i, 0))


def _full_spec(rows, cols):
    return pl.BlockSpec((rows, cols), lambda i: (0, 0))


_ZSHAPE = (
    jax.ShapeDtypeStruct((N_PAD, DH), jnp.float32),
    jax.ShapeDtypeStruct((N_PAD, DH), jnp.float32),
)


def _scale_matmul(x, w, dis_col):
    return pl.pallas_call(
        _k_scale_matmul,
        grid=(_GRID,),
        in_specs=[_row_spec(D), _full_spec(D, D), _row_spec(1)],
        out_specs=(_row_spec(DH), _row_spec(DH)),
        out_shape=_ZSHAPE,
    )(x, w, dis_col)


def _mid(p, z0, z1, dis_col, s_row, u_row, w):
    return pl.pallas_call(
        _k_mid,
        grid=(_GRID,),
        in_specs=[
            _row_spec(DH),
            _p1_spec(),
            _row_spec(DH),
            _row_spec(DH),
            _row_spec(1),
            _full_spec(1, D),
            _full_spec(1, D),
            _full_spec(D, D),
        ],
        out_specs=(_row_spec(DH), _row_spec(DH)),
        out_shape=_ZSHAPE,
    )(p, p, z0, z1, dis_col, s_row, u_row, w)


def _final(p, z0, z1, dis_col, s_row, u_row, w, b_row):
    return pl.pallas_call(
        _k_final,
        grid=(_GRID,),
        in_specs=[
            _row_spec(DH),
            _p1_spec(),
            _row_spec(DH),
            _row_spec(DH),
            _row_spec(1),
            _full_spec(1, D),
            _full_spec(1, D),
            _full_spec(D, D_OUT),
            _full_spec(1, D_OUT),
        ],
        out_specs=pl.BlockSpec((_RB, D_OUT), lambda i: (i, 0)),
        out_shape=jax.ShapeDtypeStruct((N_PAD, D_OUT), jnp.float32),
    )(p, p, z0, z1, dis_col, s_row, u_row, w, b_row)


@jax.jit
def kernel(x, edge_index, W1, b1, g1, be1, rm1, rv1, W2, b2, g2, be2, rm2, rv2,
           Wro, bro):
    pad_e = E_PAD - E
    src_p = jnp.concatenate([edge_index[0], jnp.zeros((pad_e,), jnp.int32)])
    dst_p = jnp.concatenate(
        [edge_index[1], jnp.full((pad_e,), PAD_DST, jnp.int32)]
    )
    src2d = src_p.reshape(E_PAD // CHUNK, CHUNK)
    dst2d = dst_p.reshape(E_PAD // CHUNK, CHUNK)
    x_p = jnp.concatenate([x, jnp.zeros((N_PAD - N, D), jnp.float32)])

    s1 = g1 * lax.rsqrt(rv1 + 1e-5)
    u1 = (b1 - rm1) * s1 + be1
    s2 = g2 * lax.rsqrt(rv2 + 1e-5)
    u2 = (b2 - rm2) * s2 + be2
    s1r, u1r = s1[None, :], u1[None, :]
    s2r, u2r = s2[None, :], u2[None, :]
    bror = bro[None, :]

    degp = _deg_sc(dst2d).reshape(NC, N_PAD)
    dis_col = lax.rsqrt(degp[0] + degp[1] + 1.0)[:, None]

    z1lo, z1hi = _scale_matmul(x_p, W1, dis_col)
    p1_ = _agg_sc(z1lo, z1hi, src2d, dst2d)

    z2lo, z2hi = _mid(p1_, z1lo, z1hi, dis_col, s1r, u1r, W2)
    p2_ = _agg_sc(z2lo, z2hi, src2d, dst2d)

    out = _final(p2_, z2lo, z2hi, dis_col, s2r, u2r, Wro, bror)
    return out[:N]


# speedup vs baseline: 1.1098x; 1.0076x over previous
import functools

import jax
import jax.numpy as jnp
from jax import lax
from jax.experimental import pallas as pl
from jax.experimental.pallas import tpu as pltpu
from jax.experimental.pallas import tpu_sc as plsc

N = 10000
E = 320000
D = 128
D_OUT = 64

N_PAD = 10240
CHUNK = 128
NC = 2
NS = 16
NW = NC * NS
E_PAD = N_PAD * NW
ROWS_PER_S = N_PAD // NS
PAD_DST = N + 100
NB = 5
DH = D // NC

DEG_CHUNKS = E_PAD // CHUNK // NW
DEG_GROUPS = DEG_CHUNKS // NB

AGG_CHUNKS = E_PAD // CHUNK // NS
AGG_PASSES = 2
PASS_CHUNKS = AGG_CHUNKS // AGG_PASSES
AGG_NB = 2
PASS_GROUPS = PASS_CHUNKS // AGG_NB

_sc_mesh = plsc.VectorSubcoreMesh(core_axis_name="c", subcore_axis_name="s")


@functools.partial(
    pl.kernel,
    mesh=_sc_mesh,
    out_type=jax.ShapeDtypeStruct((NC * N_PAD,), jnp.float32),
    scratch_types=[
        pltpu.VMEM_SHARED((N_PAD,), jnp.float32),
        pltpu.VMEM((DEG_CHUNKS, CHUNK), jnp.int32),
        pltpu.VMEM((CHUNK,), jnp.float32),
        pltpu.VMEM((ROWS_PER_S,), jnp.float32),
    ]
    + [pltpu.SemaphoreType.DMA] * NB,
)
def _deg_sc(dst_hbm, out_hbm, acc, didx, ones, stage, *sems):
    c = lax.axis_index("c")
    s = lax.axis_index("s")
    wid = s * NC + c

    def fill(i, _):
        stage[pl.ds(i * 16, 16)] = jnp.zeros((16,), jnp.float32)
        return 0

    lax.fori_loop(0, ROWS_PER_S // 16, fill, 0)

    def fill1(i, _):
        ones[pl.ds(i * 16, 16)] = jnp.full((16,), 1.0, jnp.float32)
        return 0

    lax.fori_loop(0, CHUNK // 16, fill1, 0)

    pltpu.sync_copy(stage, acc.at[pl.ds(s * ROWS_PER_S, ROWS_PER_S)])
    pltpu.sync_copy(dst_hbm.at[pl.ds(wid * DEG_CHUNKS, DEG_CHUNKS)], didx)
    plsc.subcore_barrier()

    def scat(ch, b):
        return pltpu.async_copy(ones, acc.at[didx.at[ch]], sems[b], add=True)

    for b in range(NB):
        scat(b, b)

    def body(k, _):
        for b in range(NB):
            pltpu.make_async_copy(ones, acc.at[didx.at[0]], sems[b]).wait()
            scat((k + 1) * NB + b, b)
        return 0

    lax.fori_loop(0, DEG_GROUPS - 1, body, 0)
    for b in range(NB):
        pltpu.make_async_copy(ones, acc.at[didx.at[0]], sems[b]).wait()
    plsc.subcore_barrier()

    pltpu.sync_copy(acc.at[pl.ds(s * ROWS_PER_S, ROWS_PER_S)], stage)
    pltpu.sync_copy(stage, out_hbm.at[pl.ds(c * N_PAD + s * ROWS_PER_S, ROWS_PER_S)])


@functools.partial(
    pl.kernel,
    mesh=_sc_mesh,
    out_type=jax.ShapeDtypeStruct((NC * N_PAD, DH), jnp.float32),
    compiler_params=pltpu.CompilerParams(use_tc_tiling_on_sc=False),
    scratch_types=[
        pltpu.VMEM_SHARED((N_PAD, DH), jnp.float32),
        pltpu.VMEM_SHARED((N_PAD, DH), jnp.float32),
        pltpu.VMEM((PASS_CHUNKS, CHUNK), jnp.int32),
        pltpu.VMEM((PASS_CHUNKS, CHUNK), jnp.int32),
    ]
    + [pltpu.VMEM((CHUNK, DH), jnp.float32)] * AGG_NB
    + [pltpu.SemaphoreType.DMA] * (2 * AGG_NB + 1),
)
def _agg_sc(zlo_hbm, zhi_hbm, src_hbm, dst_hbm, out_hbm, acc, zsp, sidx, didx,
            *rest):
    gbuf = rest[:AGG_NB]
    gsem = rest[AGG_NB : 2 * AGG_NB]
    ssem = rest[2 * AGG_NB : 3 * AGG_NB]
    stsem = rest[3 * AGG_NB]
    c = lax.axis_index("c")
    s = lax.axis_index("s")

    @pl.when(c == 0)
    def _():
        pltpu.async_copy(
            zlo_hbm.at[pl.ds(s * ROWS_PER_S, ROWS_PER_S)],
            zsp.at[pl.ds(s * ROWS_PER_S, ROWS_PER_S)],
            stsem,
        )

    @pl.when(c == 1)
    def _():
        pltpu.async_copy(
            zhi_hbm.at[pl.ds(s * ROWS_PER_S, ROWS_PER_S)],
            zsp.at[pl.ds(s * ROWS_PER_S, ROWS_PER_S)],
            stsem,
        )

    def fillz(i, _):
        gbuf[0][i // (DH // 16), pl.ds((i % (DH // 16)) * 16, 16)] = jnp.zeros(
            (16,), jnp.float32
        )
        return 0

    lax.fori_loop(0, CHUNK * DH // 16, fillz, 0)

    def zrow(j, _):
        pltpu.sync_copy(gbuf[0], acc.at[pl.ds(s * ROWS_PER_S + j * CHUNK, CHUNK)])
        return 0

    lax.fori_loop(0, ROWS_PER_S // CHUNK, zrow, 0)
    pltpu.make_async_copy(
        zlo_hbm.at[pl.ds(0, ROWS_PER_S)],
        zsp.at[pl.ds(0, ROWS_PER_S)],
        stsem,
    ).wait()
    plsc.subcore_barrier()

    def gat(ch, b):
        return pltpu.async_copy(zsp.at[sidx.at[ch]], gbuf[b], gsem[b])

    def gat_wait(b):
        pltpu.make_async_copy(zsp.at[sidx.at[0]], gbuf[b], gsem[b]).wait()

    def scat(ch, b):
        return pltpu.async_copy(gbuf[b], acc.at[didx.at[ch]], ssem[b], add=True)

    def scat_wait(b):
        pltpu.make_async_copy(gbuf[b], acc.at[didx.at[0]], ssem[b]).wait()

    for p in range(AGG_PASSES):
        base = s * AGG_CHUNKS + p * PASS_CHUNKS
        pltpu.sync_copy(src_hbm.at[pl.ds(base, PASS_CHUNKS)], sidx)
        pltpu.sync_copy(dst_hbm.at[pl.ds(base, PASS_CHUNKS)], didx)

        for b in range(AGG_NB):
            gat(b, b)

        def body(k, _):
            for b in range(AGG_NB):
                gat_wait(b)
                scat(k * AGG_NB + b, b)
            for b in range(AGG_NB):
                scat_wait(b)
                gat((k + 1) * AGG_NB + b, b)
            return 0

        lax.fori_loop(0, PASS_GROUPS - 1, body, 0)

        for b in range(AGG_NB):
            gat_wait(b)
            scat((PASS_GROUPS - 1) * AGG_NB + b, b)
        for b in range(AGG_NB):
            scat_wait(b)
    plsc.subcore_barrier()

    def crow(j, _):
        row = s * ROWS_PER_S + j * CHUNK
        pltpu.sync_copy(acc.at[pl.ds(row, CHUNK)], gbuf[0])
        pltpu.sync_copy(gbuf[0], out_hbm.at[pl.ds(c * N_PAD + row, CHUNK)])
        return 0

    lax.fori_loop(0, ROWS_PER_S // CHUNK, crow, 0)


_RB = 2560
_GRID = N_PAD // _RB


def _k_scale_matmul(x_ref, w_ref, dis_ref, o0_ref, o1_ref):
    z = dis_ref[...] * jnp.dot(
        x_ref[...], w_ref[...], preferred_element_type=jnp.float32
    )
    o0_ref[...] = z[:, :DH]
    o1_ref[...] = z[:, DH:]


def _k_mid(p0_ref, p1_ref, z0_ref, z1_ref, dis_ref, s_ref, u_ref, w_ref,
           o0_ref, o1_ref):
    dis = dis_ref[...]
    p = jnp.concatenate([p0_ref[...], p1_ref[...]], axis=1)
    z = jnp.concatenate([z0_ref[...], z1_ref[...]], axis=1)
    h = jnp.maximum(dis * (p + z) * s_ref[...] + u_ref[...], 0.0)
    zn = dis * jnp.dot(h, w_ref[...], preferred_element_type=jnp.float32)
    o0_ref[...] = zn[:, :DH]
    o1_ref[...] = zn[:, DH:]


def _k_final(p0_ref, p1_ref, z0_ref, z1_ref, dis_ref, s_ref, u_ref, w_ref,
             b_ref, o_ref):
    dis = dis_ref[...]
    p = jnp.concatenate([p0_ref[...], p1_ref[...]], axis=1)
    z = jnp.concatenate([z0_ref[...], z1_ref[...]], axis=1)
    h = jnp.maximum(dis * (p + z) * s_ref[...] + u_ref[...], 0.0)
    o_ref[...] = (
        jnp.dot(h, w_ref[...], preferred_element_type=jnp.float32) + b_ref[...]
    )


def _row_spec(cols):
    return pl.BlockSpec((_RB, cols), lambda i: (i, 0))


def _p1_spec():
    return pl.BlockSpec((_RB, DH), lambda i: (_GRID + i, 0))


def _full_spec(rows, cols):
    return pl.BlockSpec((rows, cols), lambda i: (0, 0))


_ZSHAPE = (
    jax.ShapeDtypeStruct((N_PAD, DH), jnp.float32),
    jax.ShapeDtypeStruct((N_PAD, DH), jnp.float32),
)


def _scale_matmul(x, w, dis_col):
    return pl.pallas_call(
        _k_scale_matmul,
        grid=(_GRID,),
        in_specs=[_row_spec(D), _full_spec(D, D), _row_spec(1)],
        out_specs=(_row_spec(DH), _row_spec(DH)),
        out_shape=_ZSHAPE,
    )(x, w, dis_col)


def _mid(p, z0, z1, dis_col, s_row, u_row, w):
    return pl.pallas_call(
        _k_mid,
        grid=(_GRID,),
        in_specs=[
            _row_spec(DH),
            _p1_spec(),
            _row_spec(DH),
            _row_spec(DH),
            _row_spec(1),
            _full_spec(1, D),
            _full_spec(1, D),
            _full_spec(D, D),
        ],
        out_specs=(_row_spec(DH), _row_spec(DH)),
        out_shape=_ZSHAPE,
    )(p, p, z0, z1, dis_col, s_row, u_row, w)


def _final(p, z0, z1, dis_col, s_row, u_row, w, b_row):
    return pl.pallas_call(
        _k_final,
        grid=(_GRID,),
        in_specs=[
            _row_spec(DH),
            _p1_spec(),
            _row_spec(DH),
            _row_spec(DH),
            _row_spec(1),
            _full_spec(1, D),
            _full_spec(1, D),
            _full_spec(D, D_OUT),
            _full_spec(1, D_OUT),
        ],
        out_specs=pl.BlockSpec((_RB, D_OUT), lambda i: (i, 0)),
        out_shape=jax.ShapeDtypeStruct((N_PAD, D_OUT), jnp.float32),
    )(p, p, z0, z1, dis_col, s_row, u_row, w, b_row)


@jax.jit
def kernel(x, edge_index, W1, b1, g1, be1, rm1, rv1, W2, b2, g2, be2, rm2, rv2,
           Wro, bro):
    pad_e = E_PAD - E
    src_p = jnp.concatenate([edge_index[0], jnp.zeros((pad_e,), jnp.int32)])
    dst_p = jnp.concatenate(
        [edge_index[1], jnp.full((pad_e,), PAD_DST, jnp.int32)]
    )
    src2d = src_p.reshape(E_PAD // CHUNK, CHUNK)
    dst2d = dst_p.reshape(E_PAD // CHUNK, CHUNK)
    x_p = jnp.concatenate([x, jnp.zeros((N_PAD - N, D), jnp.float32)])

    s1 = g1 * lax.rsqrt(rv1 + 1e-5)
    u1 = (b1 - rm1) * s1 + be1
    s2 = g2 * lax.rsqrt(rv2 + 1e-5)
    u2 = (b2 - rm2) * s2 + be2
    s1r, u1r = s1[None, :], u1[None, :]
    s2r, u2r = s2[None, :], u2[None, :]
    bror = bro[None, :]

    degp = _deg_sc(dst2d).reshape(NC, N_PAD)
    dis_col = lax.rsqrt(degp[0] + degp[1] + 1.0)[:, None]

    z1lo, z1hi = _scale_matmul(x_p, W1, dis_col)
    p1_ = _agg_sc(z1lo, z1hi, src2d, dst2d)

    z2lo, z2hi = _mid(p1_, z1lo, z1hi, dis_col, s1r, u1r, W2)
    p2_ = _agg_sc(z2lo, z2hi, src2d, dst2d)

    out = _final(p2_, z2lo, z2hi, dis_col, s2r, u2r, Wro, bror)
    return out[:N]


# speedup vs baseline: 1.1134x; 1.0032x over previous
import functools

import jax
import jax.numpy as jnp
from jax import lax
from jax.experimental import pallas as pl
from jax.experimental.pallas import tpu as pltpu
from jax.experimental.pallas import tpu_sc as plsc

N = 10000
E = 320000
D = 128
D_OUT = 64

N_PAD = 10240
CHUNK = 128
NC = 2
NS = 16
NW = NC * NS
E_PAD = N_PAD * NW
ROWS_PER_S = N_PAD // NS
PAD_DST = N + 100
NB = 5
DH = D // NC

DEG_CHUNKS = E_PAD // CHUNK // NW
DEG_GROUPS = DEG_CHUNKS // NB

AGG_CHUNKS = E_PAD // CHUNK // NS
AGG_PASSES = 2
PASS_CHUNKS = AGG_CHUNKS // AGG_PASSES
AGG_NB = 2
PASS_GROUPS = PASS_CHUNKS // AGG_NB

_sc_mesh = plsc.VectorSubcoreMesh(core_axis_name="c", subcore_axis_name="s")


@functools.partial(
    pl.kernel,
    mesh=_sc_mesh,
    out_type=jax.ShapeDtypeStruct((NC * N_PAD,), jnp.float32),
    scratch_types=[
        pltpu.VMEM_SHARED((N_PAD,), jnp.float32),
        pltpu.VMEM((DEG_CHUNKS, CHUNK), jnp.int32),
        pltpu.VMEM((CHUNK,), jnp.float32),
        pltpu.VMEM((ROWS_PER_S,), jnp.float32),
    ]
    + [pltpu.SemaphoreType.DMA] * NB,
)
def _deg_sc(dst_hbm, out_hbm, acc, didx, ones, stage, *sems):
    c = lax.axis_index("c")
    s = lax.axis_index("s")
    wid = s * NC + c

    def fill(i, _):
        stage[pl.ds(i * 16, 16)] = jnp.zeros((16,), jnp.float32)
        return 0

    lax.fori_loop(0, ROWS_PER_S // 16, fill, 0)

    def fill1(i, _):
        ones[pl.ds(i * 16, 16)] = jnp.full((16,), 1.0, jnp.float32)
        return 0

    lax.fori_loop(0, CHUNK // 16, fill1, 0)

    pltpu.sync_copy(stage, acc.at[pl.ds(s * ROWS_PER_S, ROWS_PER_S)])
    pltpu.sync_copy(dst_hbm.at[pl.ds(wid * DEG_CHUNKS, DEG_CHUNKS)], didx)
    plsc.subcore_barrier()

    def scat(ch, b):
        return pltpu.async_copy(ones, acc.at[didx.at[ch]], sems[b], add=True)

    for b in range(NB):
        scat(b, b)

    def body(k, _):
        for b in range(NB):
            pltpu.make_async_copy(ones, acc.at[didx.at[0]], sems[b]).wait()
            scat((k + 1) * NB + b, b)
        return 0

    lax.fori_loop(0, DEG_GROUPS - 1, body, 0)
    for b in range(NB):
        pltpu.make_async_copy(ones, acc.at[didx.at[0]], sems[b]).wait()
    plsc.subcore_barrier()

    pltpu.sync_copy(acc.at[pl.ds(s * ROWS_PER_S, ROWS_PER_S)], stage)
    pltpu.sync_copy(stage, out_hbm.at[pl.ds(c * N_PAD + s * ROWS_PER_S, ROWS_PER_S)])


@functools.partial(
    pl.kernel,
    mesh=_sc_mesh,
    out_type=jax.ShapeDtypeStruct((NC * N_PAD, DH), jnp.float32),
    compiler_params=pltpu.CompilerParams(use_tc_tiling_on_sc=False),
    scratch_types=[
        pltpu.VMEM_SHARED((N_PAD, DH), jnp.float32),
        pltpu.VMEM_SHARED((N_PAD, DH), jnp.float32),
        pltpu.VMEM((PASS_CHUNKS, CHUNK), jnp.int32),
        pltpu.VMEM((PASS_CHUNKS, CHUNK), jnp.int32),
    ]
    + [pltpu.VMEM((CHUNK, DH), jnp.float32)] * AGG_NB
    + [pltpu.SemaphoreType.DMA] * (2 * AGG_NB + 1),
)
def _agg_sc(zlo_hbm, zhi_hbm, src_hbm, dst_hbm, out_hbm, acc, zsp, sidx, didx,
            *rest):
    gbuf = rest[:AGG_NB]
    gsem = rest[AGG_NB : 2 * AGG_NB]
    ssem = rest[2 * AGG_NB : 3 * AGG_NB]
    stsem = rest[3 * AGG_NB]
    c = lax.axis_index("c")
    s = lax.axis_index("s")

    @pl.when(c == 0)
    def _():
        pltpu.async_copy(
            zlo_hbm.at[pl.ds(s * ROWS_PER_S, ROWS_PER_S)],
            zsp.at[pl.ds(s * ROWS_PER_S, ROWS_PER_S)],
            stsem,
        )

    @pl.when(c == 1)
    def _():
        pltpu.async_copy(
            zhi_hbm.at[pl.ds(s * ROWS_PER_S, ROWS_PER_S)],
            zsp.at[pl.ds(s * ROWS_PER_S, ROWS_PER_S)],
            stsem,
        )

    def fillz(i, _):
        gbuf[0][i // (DH // 16), pl.ds((i % (DH // 16)) * 16, 16)] = jnp.zeros(
            (16,), jnp.float32
        )
        return 0

    lax.fori_loop(0, CHUNK * DH // 16, fillz, 0)

    def zrow(j, _):
        pltpu.sync_copy(gbuf[0], acc.at[pl.ds(s * ROWS_PER_S + j * CHUNK, CHUNK)])
        return 0

    lax.fori_loop(0, ROWS_PER_S // CHUNK, zrow, 0)
    pltpu.make_async_copy(
        zlo_hbm.at[pl.ds(0, ROWS_PER_S)],
        zsp.at[pl.ds(0, ROWS_PER_S)],
        stsem,
    ).wait()
    plsc.subcore_barrier()

    def gat(ch, b):
        return pltpu.async_copy(zsp.at[sidx.at[ch]], gbuf[b], gsem[b])

    def gat_wait(b):
        pltpu.make_async_copy(zsp.at[sidx.at[0]], gbuf[b], gsem[b]).wait()

    def scat(ch, b):
        return pltpu.async_copy(gbuf[b], acc.at[didx.at[ch]], ssem[b], add=True)

    def scat_wait(b):
        pltpu.make_async_copy(gbuf[b], acc.at[didx.at[0]], ssem[b]).wait()

    for p in range(AGG_PASSES):
        base = s * AGG_CHUNKS + p * PASS_CHUNKS
        pltpu.sync_copy(src_hbm.at[pl.ds(base, PASS_CHUNKS)], sidx)
        pltpu.sync_copy(dst_hbm.at[pl.ds(base, PASS_CHUNKS)], didx)

        for b in range(AGG_NB):
            gat(b, b)

        def body(k, _):
            for b in range(AGG_NB):
                gat_wait(b)
                scat(k * AGG_NB + b, b)
            for b in range(AGG_NB):
                scat_wait(b)
                gat((k + 1) * AGG_NB + b, b)
            return 0

        lax.fori_loop(0, PASS_GROUPS - 1, body, 0)

        for b in range(AGG_NB):
            gat_wait(b)
            scat((PASS_GROUPS - 1) * AGG_NB + b, b)
        for b in range(AGG_NB):
            scat_wait(b)
    plsc.subcore_barrier()

    def crow(j, _):
        row = s * ROWS_PER_S + j * CHUNK
        pltpu.sync_copy(acc.at[pl.ds(row, CHUNK)], gbuf[0])
        pltpu.sync_copy(gbuf[0], out_hbm.at[pl.ds(c * N_PAD + row, CHUNK)])
        return 0

    lax.fori_loop(0, ROWS_PER_S // CHUNK, crow, 0)


_RB = 5120
_GRID = N_PAD // _RB


def _k_scale_matmul(x_ref, w_ref, dis_ref, o0_ref, o1_ref):
    z = dis_ref[...] * jnp.dot(
        x_ref[...], w_ref[...], preferred_element_type=jnp.float32
    )
    o0_ref[...] = z[:, :DH]
    o1_ref[...] = z[:, DH:]


def _k_mid(p0_ref, p1_ref, z0_ref, z1_ref, dis_ref, s_ref, u_ref, w_ref,
           o0_ref, o1_ref):
    dis = dis_ref[...]
    p = jnp.concatenate([p0_ref[...], p1_ref[...]], axis=1)
    z = jnp.concatenate([z0_ref[...], z1_ref[...]], axis=1)
    h = jnp.maximum(dis * (p + z) * s_ref[...] + u_ref[...], 0.0)
    zn = dis * jnp.dot(h, w_ref[...], preferred_element_type=jnp.float32)
    o0_ref[...] = zn[:, :DH]
    o1_ref[...] = zn[:, DH:]


def _k_final(p0_ref, p1_ref, z0_ref, z1_ref, dis_ref, s_ref, u_ref, w_ref,
             b_ref, o_ref):
    dis = dis_ref[...]
    p = jnp.concatenate([p0_ref[...], p1_ref[...]], axis=1)
    z = jnp.concatenate([z0_ref[...], z1_ref[...]], axis=1)
    h = jnp.maximum(dis * (p + z) * s_ref[...] + u_ref[...], 0.0)
    o_ref[...] = (
        jnp.dot(h, w_ref[...], preferred_element_type=jnp.float32) + b_ref[...]
    )


def _row_spec(cols):
    return pl.BlockSpec((_RB, cols), lambda i: (i, 0))


def _p1_spec():
    return pl.BlockSpec((_RB, DH), lambda i: (_GRID + i, 0))


def _full_spec(rows, cols):
    return pl.BlockSpec((rows, cols), lambda i: (0, 0))


_ZSHAPE = (
    jax.ShapeDtypeStruct((N_PAD, DH), jnp.float32),
    jax.ShapeDtypeStruct((N_PAD, DH), jnp.float32),
)


def _scale_matmul(x, w, dis_col):
    return pl.pallas_call(
        _k_scale_matmul,
        grid=(_GRID,),
        in_specs=[_row_spec(D), _full_spec(D, D), _row_spec(1)],
        out_specs=(_row_spec(DH), _row_spec(DH)),
        out_shape=_ZSHAPE,
    )(x, w, dis_col)


def _mid(p, z0, z1, dis_col, s_row, u_row, w):
    return pl.pallas_call(
        _k_mid,
        grid=(_GRID,),
        in_specs=[
            _row_spec(DH),
            _p1_spec(),
            _row_spec(DH),
            _row_spec(DH),
            _row_spec(1),
            _full_spec(1, D),
            _full_spec(1, D),
            _full_spec(D, D),
        ],
        out_specs=(_row_spec(DH), _row_spec(DH)),
        out_shape=_ZSHAPE,
    )(p, p, z0, z1, dis_col, s_row, u_row, w)


def _final(p, z0, z1, dis_col, s_row, u_row, w, b_row):
    return pl.pallas_call(
        _k_final,
        grid=(_GRID,),
        in_specs=[
            _row_spec(DH),
            _p1_spec(),
            _row_spec(DH),
            _row_spec(DH),
            _row_spec(1),
            _full_spec(1, D),
            _full_spec(1, D),
            _full_spec(D, D_OUT),
            _full_spec(1, D_OUT),
        ],
        out_specs=pl.BlockSpec((_RB, D_OUT), lambda i: (i, 0)),
        out_shape=jax.ShapeDtypeStruct((N_PAD, D_OUT), jnp.float32),
    )(p, p, z0, z1, dis_col, s_row, u_row, w, b_row)


@jax.jit
def kernel(x, edge_index, W1, b1, g1, be1, rm1, rv1, W2, b2, g2, be2, rm2, rv2,
           Wro, bro):
    pad_e = E_PAD - E
    src_p = jnp.concatenate([edge_index[0], jnp.zeros((pad_e,), jnp.int32)])
    dst_p = jnp.concatenate(
        [edge_index[1], jnp.full((pad_e,), PAD_DST, jnp.int32)]
    )
    src2d = src_p.reshape(E_PAD // CHUNK, CHUNK)
    dst2d = dst_p.reshape(E_PAD // CHUNK, CHUNK)
    x_p = jnp.concatenate([x, jnp.zeros((N_PAD - N, D), jnp.float32)])

    s1 = g1 * lax.rsqrt(rv1 + 1e-5)
    u1 = (b1 - rm1) * s1 + be1
    s2 = g2 * lax.rsqrt(rv2 + 1e-5)
    u2 = (b2 - rm2) * s2 + be2
    s1r, u1r = s1[None, :], u1[None, :]
    s2r, u2r = s2[None, :], u2[None, :]
    bror = bro[None, :]

    degp = _deg_sc(dst2d).reshape(NC, N_PAD)
    dis_col = lax.rsqrt(degp[0] + degp[1] + 1.0)[:, None]

    z1lo, z1hi = _scale_matmul(x_p, W1, dis_col)
    p1_ = _agg_sc(z1lo, z1hi, src2d, dst2d)

    z2lo, z2hi = _mid(p1_, z1lo, z1hi, dis_col, s1r, u1r, W2)
    p2_ = _agg_sc(z2lo, z2hi, src2d, dst2d)

    out = _final(p2_, z2lo, z2hi, dis_col, s2r, u2r, Wro, bror)
    return out[:N]
